# baseline (speedup 1.0000x reference)
import functools

import jax
import jax.numpy as jnp
from jax import lax
from jax.experimental import pallas as pl
from jax.experimental.pallas import tpu as pltpu
from jax.experimental.pallas import tpu_sc as plsc

N = 10000
E = 160000
NODE_IN = 128
EDGE_IN = 16
HID = 32
EHID = 64
NMP = 6
NS2S = 6
NLSTM = 3

NC = 2
NS = 16
NW = NC * NS
TR = 128
TPW = 40
EPW = TR * TPW
EP = NW * EPW
PH = 4
TPP = TPW // PH
RPP = TR * TPP
NACC = 10240
RPS = NACC // NS

@functools.cache
def _mesh():
    return plsc.VectorSubcoreMesh(core_axis_name="c", subcore_axis_name="s",
                                  num_cores=NC, num_subcores=NS)


def _sc_gather_body(x_hbm, src_hbm, out_hbm, idx_v, rows_v, sem):
    cid = lax.axis_index("c")
    sid = lax.axis_index("s")
    wid = sid * NC + cid
    base = wid * EPW
    pltpu.sync_copy(src_hbm.at[wid], idx_v)

    def phase(ph, carry):
        descs = []
        for j in range(TPP):
            d = pltpu.async_copy(
                x_hbm.at[idx_v.at[ph * TPP + j]],
                rows_v.at[pl.ds(j * TR, TR)],
                sem,
            )
            descs.append(d)
        for d in descs:
            d.wait()
        pltpu.sync_copy(rows_v, out_hbm.at[pl.ds(base + ph * RPP, RPP)])
        return carry

    lax.fori_loop(0, PH, phase, 0)


def _sc_gather(x, src3):
    f = functools.partial(
        pl.kernel,
        out_type=jax.ShapeDtypeStruct((EP, HID), jnp.float32),
        mesh=_mesh(),
        compiler_params=pltpu.CompilerParams(use_tc_tiling_on_sc=False),
        scratch_types=[
            pltpu.VMEM((TPW, TR), jnp.int32),
            pltpu.VMEM((RPP, HID), jnp.float32),
            pltpu.SemaphoreType.DMA,
        ],
    )(_sc_gather_body)
    return f(x, src3)


def _sc_scatter_body(msg_hbm, dst_hbm, zeros_hbm, out_hbm,
                     idx_v, rows_v, stage_v, acc):
    cid = lax.axis_index("c")
    sid = lax.axis_index("s")
    wid = sid * NC + cid
    base = wid * EPW
    pltpu.sync_copy(zeros_hbm.at[pl.ds(sid * RPS, RPS)], stage_v)
    pltpu.sync_copy(stage_v, acc.at[pl.ds(sid * RPS, RPS)])
    plsc.subcore_barrier()
    pltpu.sync_copy(dst_hbm.at[wid], idx_v)

    def phase(ph, carry):
        pltpu.sync_copy(msg_hbm.at[pl.ds(base + ph * RPP, RPP)], rows_v)
        for j in range(TPP):
            pltpu.sync_copy(
                rows_v.at[pl.ds(j * TR, TR)],
                acc.at[idx_v.at[ph * TPP + j]],
                add=True,
            )
        return carry

    lax.fori_loop(0, PH, phase, 0)
    plsc.subcore_barrier()
    pltpu.sync_copy(acc.at[pl.ds(sid * RPS, RPS)], stage_v)
    pltpu.sync_copy(stage_v, out_hbm.at[cid, pl.ds(sid * RPS, RPS)])


def _sc_scatter(msg, dst3, zeros):
    f = functools.partial(
        pl.kernel,
        out_type=jax.ShapeDtypeStruct((NC, NACC, HID), jnp.float32),
        mesh=_mesh(),
        compiler_params=pltpu.CompilerParams(use_tc_tiling_on_sc=False),
        scratch_types=[
            pltpu.VMEM((TPW, TR), jnp.int32),
            pltpu.VMEM((RPP, HID), jnp.float32),
            pltpu.VMEM((RPS, HID), jnp.float32),
            pltpu.VMEM_SHARED((NACC, HID), jnp.float32),
        ],
    )(_sc_scatter_body)
    return f(msg, dst3, zeros)


BE = 640


def _msg_body(e_ref, xs_ref, w1t_ref, b1_ref, w2t_ref, b2_ref, o_ref):
    r = jnp.dot(e_ref[...], w1t_ref[...], preferred_element_type=jnp.float32)
    r = jnp.maximum(r + b1_ref[...], 0.0)
    w = jnp.dot(r, w2t_ref[...], preferred_element_type=jnp.float32)
    w = w + b2_ref[...]
    x = xs_ref[...]
    acc = x[:, 0:1] * w[:, 0:HID]
    for i in range(1, HID):
        acc = acc + x[:, i:i + 1] * w[:, i * HID:(i + 1) * HID]
    o_ref[...] = acc


def _msg(e_pad, x_src, w1t, b1, w2t, b2):
    grid = EP // BE
    return pl.pallas_call(
        _msg_body,
        grid=(grid,),
        in_specs=[
            pl.BlockSpec((BE, EDGE_IN), lambda i: (i, 0)),
            pl.BlockSpec((BE, HID), lambda i: (i, 0)),
            pl.BlockSpec((EDGE_IN, EHID), lambda i: (0, 0)),
            pl.BlockSpec((1, EHID), lambda i: (0, 0)),
            pl.BlockSpec((EHID, HID * HID), lambda i: (0, 0)),
            pl.BlockSpec((1, HID * HID), lambda i: (0, 0)),
        ],
        out_specs=pl.BlockSpec((BE, HID), lambda i: (i, 0)),
        out_shape=jax.ShapeDtypeStruct((EP, HID), jnp.float32),
    )(e_pad, x_src, w1t, b1, w2t, b2)


def _dot(a, b):
    return jnp.dot(a, b, preferred_element_type=jnp.float32,
                   precision=lax.Precision.HIGHEST)


def _gru_body(m0_ref, m1_ref, h_ref,
              wir_ref, wiz_ref, win_ref, whr_ref, whz_ref, whn_ref,
              br_ref, bz_ref, bin_ref, bhn_ref, cb_ref, o_ref):
    m = m0_ref[...] + m1_ref[...] + cb_ref[...]
    h = h_ref[...]
    r = jax.nn.sigmoid(_dot(m, wir_ref[...]) + _dot(h, whr_ref[...]) + br_ref[...])
    z = jax.nn.sigmoid(_dot(m, wiz_ref[...]) + _dot(h, whz_ref[...]) + bz_ref[...])
    hn = _dot(h, whn_ref[...]) + bhn_ref[...]
    n = jnp.tanh(_dot(m, win_ref[...]) + bin_ref[...] + r * hn)
    o_ref[...] = (1.0 - z) * n + z * h


GRB = 2000


def _gru(m0, m1, h, gw):
    row = pl.BlockSpec((GRB, HID), lambda i: (i, 0))
    return pl.pallas_call(
        _gru_body,
        grid=(N // GRB,),
        in_specs=[row] * 3
                 + [pl.BlockSpec((HID, HID), lambda i: (0, 0))] * 6
                 + [pl.BlockSpec((1, HID), lambda i: (0, 0))] * 5,
        out_specs=row,
        out_shape=jax.ShapeDtypeStruct((N, HID), jnp.float32),
    )(m0, m1, h, *gw)


def _lin0_body(x_ref, w_ref, b_ref, o_ref):
    o_ref[...] = jnp.dot(x_ref[...], w_ref[...],
                         preferred_element_type=jnp.float32) + b_ref[...]


def _lin0(n_feat, w0t, b0):
    return pl.pallas_call(
        _lin0_body,
        grid=(N // GRB,),
        in_specs=[pl.BlockSpec((GRB, NODE_IN), lambda i: (i, 0)),
                  pl.BlockSpec((NODE_IN, HID), lambda i: (0, 0)),
                  pl.BlockSpec((1, HID), lambda i: (0, 0))],
        out_specs=pl.BlockSpec((GRB, HID), lambda i: (i, 0)),
        out_shape=jax.ShapeDtypeStruct((N, HID), jnp.float32),
    )(n_feat, w0t, b0)


def _s2s_body(out_ref, *refs):
    lw = refs[:4 * NLSTM]
    w3t_ref, b3_ref, wpt_ref, bp_ref, pred_ref = refs[4 * NLSTM:]
    out = out_ref[...]
    hs = [jnp.zeros((1, HID), jnp.float32) for _ in range(NLSTM)]
    cs = [jnp.zeros((1, HID), jnp.float32) for _ in range(NLSTM)]
    q_star = jnp.zeros((1, 2 * HID), jnp.float32)
    for _ in range(NS2S):
        x = q_star
        for l in range(NLSTM):
            wiht, whht, bi, bh = lw[4 * l:4 * l + 4]
            g = (jnp.dot(x, wiht[...], preferred_element_type=jnp.float32,
                 precision=lax.Precision.HIGHEST)
                 + jnp.dot(hs[l], whht[...], preferred_element_type=jnp.float32,
                 precision=lax.Precision.HIGHEST)
                 + bi[...] + bh[...])
            i = jax.nn.sigmoid(g[:, 0 * HID:1 * HID])
            f = jax.nn.sigmoid(g[:, 1 * HID:2 * HID])
            gg = jnp.tanh(g[:, 2 * HID:3 * HID])
            o = jax.nn.sigmoid(g[:, 3 * HID:4 * HID])
            cs[l] = f * cs[l] + i * gg
            hs[l] = o * jnp.tanh(cs[l])
            x = hs[l]
        q = x
        e = jnp.sum(out * q, axis=-1, keepdims=True)
        mx = jnp.max(e)
        a = jnp.exp(e - mx)
        s = jnp.sum(a)
        readout = jnp.sum(out * a, axis=0, keepdims=True) / s
        q_star = jnp.concatenate([q, readout], axis=-1)
    z = jnp.maximum(jnp.dot(q_star, w3t_ref[...],
                            preferred_element_type=jnp.float32,
                 precision=lax.Precision.HIGHEST) + b3_ref[...], 0.0)
    pred_ref[...] = jnp.dot(z, wpt_ref[...],
                            preferred_element_type=jnp.float32,
                 precision=lax.Precision.HIGHEST) + bp_ref[...]


def _s2s(out, lw_flat, w3t, b3, wpt, bp):
    full = lambda s: pl.BlockSpec(s, lambda: tuple(0 for _ in s))
    args = [out] + lw_flat + [w3t, b3, wpt, bp]
    return pl.pallas_call(
        _s2s_body,
        in_specs=[full(tuple(a.shape)) for a in args],
        out_specs=full((1, 1)),
        out_shape=jax.ShapeDtypeStruct((1, 1), jnp.float32),
    )(*args)


def kernel(n_feat, e_feat, edge_index, params):
    p = params
    src = edge_index[0]
    dst = edge_index[1]
    src3 = jnp.concatenate(
        [src, jnp.zeros((EP - E,), jnp.int32)]).reshape(NW, TPW, TR)
    dst3 = jnp.concatenate(
        [dst, jnp.full((EP - E,), N, jnp.int32)]).reshape(NW, TPW, TR)
    e_pad = jnp.concatenate(
        [e_feat, jnp.zeros((EP - E, EDGE_IN), jnp.float32)], axis=0)
    zeros = jnp.zeros((NACC, HID), jnp.float32)

    w0t = p['W0'].T
    b0 = p['b0'][None, :]
    w1t = p['We1'].T
    b1 = p['be1'][None, :]
    w2t = p['We2'].T
    b2 = p['be2'][None, :]

    wiht = p['gru_Wih'].T
    whht = p['gru_Whh'].T
    gw = [wiht[:, 0:HID], wiht[:, HID:2 * HID], wiht[:, 2 * HID:3 * HID],
          whht[:, 0:HID], whht[:, HID:2 * HID], whht[:, 2 * HID:3 * HID],
          (p['gru_bih'][0:HID] + p['gru_bhh'][0:HID])[None, :],
          (p['gru_bih'][HID:2 * HID] + p['gru_bhh'][HID:2 * HID])[None, :],
          p['gru_bih'][2 * HID:3 * HID][None, :],
          p['gru_bhh'][2 * HID:3 * HID][None, :],
          p['conv_b'][None, :]]

    lw_flat = []
    for l in range(NLSTM):
        lw_flat += [p['lWih%d' % l].T, p['lWhh%d' % l].T,
                    p['lbih%d' % l][None, :], p['lbhh%d' % l][None, :]]

    out = _lin0(n_feat, w0t, b0)
    h = out
    for _ in range(NMP):
        x_src = _sc_gather(out, src3)
        msg = _msg(e_pad, x_src, w1t, b1, w2t, b2)
        parts = _sc_scatter(msg, dst3, zeros)
        out = _gru(parts[0, :N], parts[1, :N], h, gw)
        h = out
    predict = _s2s(out, lw_flat, p['W3'].T, p['b3'][None, :],
                   p['Wp'].T, p['bp'][None, :])
    return h[None, :, :], predict
